# baseline (speedup 1.0000x reference)
import functools

import jax
import jax.numpy as jnp
from jax import lax
from jax.experimental import pallas as pl
from jax.experimental.pallas import tpu as pltpu
from jax.experimental.pallas import tpu_sc as plsc

_N = 10000
_E = 160000
_D = 256

_NC = 2
_NS = 16
_NW = _NC * _NS
_L = 16

_NPAD = 10240
_EPAD = 163840
_TE_A = _EPAD // _NW
_TE_B = _EPAD // _NS
_G = 128
_NCH = _TE_B // _G
_CG = 4
_DG = _D // _CG
_RPT = _NPAD // _NS

_mesh = plsc.VectorSubcoreMesh(core_axis_name="c", subcore_axis_name="s")


@functools.partial(
    pl.kernel,
    out_type=jax.ShapeDtypeStruct((_NW, _NPAD), jnp.float32),
    mesh=_mesh,
    compiler_params=pltpu.CompilerParams(needs_layout_passes=False),
    scratch_types=[
        pltpu.VMEM((_NPAD,), jnp.float32),
        pltpu.VMEM((_TE_A,), jnp.int32),
    ],
)
def _hist_kernel(dst_hbm, out_hbm, hist_v, dst_v):
    c = lax.axis_index("c")
    s = lax.axis_index("s")
    wid = c * _NS + s

    zeros16 = jnp.zeros((_L,), jnp.float32)

    def zbody(i, carry):
        hist_v[pl.ds(i * _L, _L)] = zeros16
        return carry

    lax.fori_loop(0, _NPAD // _L, zbody, 0)

    pltpu.sync_copy(dst_hbm.at[pl.ds(wid * _TE_A, _TE_A)], dst_v)

    ones16 = jnp.ones((_L,), jnp.float32)

    def body(i, carry):
        idx = dst_v[pl.ds(i * _L, _L)]
        plsc.addupdate_scatter(hist_v, [idx], ones16)
        return carry

    lax.fori_loop(0, _TE_A // _L, body, 0)

    pltpu.sync_copy(hist_v, out_hbm.at[wid])


@functools.partial(
    pl.kernel,
    out_type=jax.ShapeDtypeStruct((_NPAD, _D), jnp.float32),
    mesh=_mesh,
    compiler_params=pltpu.CompilerParams(needs_layout_passes=False,
                                         use_tc_tiling_on_sc=False),
    scratch_types=[
        pltpu.VMEM((_TE_B,), jnp.int32),
        pltpu.VMEM((_TE_B,), jnp.int32),
        pltpu.VMEM((_TE_B,), jnp.float32),
        pltpu.VMEM((2, _G, _DG), jnp.float32),
        pltpu.VMEM((2, _G, _DG), jnp.float32),
        pltpu.VMEM((_G,), jnp.float32),
        pltpu.VMEM((2, _G), jnp.int32),
        pltpu.VMEM_SHARED((_NPAD, _DG), jnp.float32),
        pltpu.SemaphoreType.DMA,
        pltpu.SemaphoreType.DMA,
        pltpu.SemaphoreType.DMA,
        pltpu.SemaphoreType.DMA,
    ],
)
def _agg_kernel(feat_hbm, src_hbm, dst_hbm, w_hbm, zeros_hbm, out_hbm,
                src_v, dst_v, w_v, rows_v, sbuf_v, cs_v, loc_v, acc_sh,
                gsem0, gsem1, ssem0, ssem1):
    c = lax.axis_index("c")
    s = lax.axis_index("s")
    base_e = s * _TE_B
    gsems = (gsem0, gsem1)
    ssems = (ssem0, ssem1)

    pltpu.sync_copy(src_hbm.at[pl.ds(base_e, _TE_B)], src_v)
    pltpu.sync_copy(dst_hbm.at[pl.ds(base_e, _TE_B)], dst_v)
    pltpu.sync_copy(w_hbm.at[pl.ds(base_e, _TE_B)], w_v)

    for p in range(2):
        cg = c * 2 + p
        col = cg * _DG

        pltpu.sync_copy(zeros_hbm, acc_sh.at[pl.ds(s * _RPT, _RPT)])
        plsc.subcore_barrier()

        def fire_gather(k, b):
            pltpu.async_copy(
                feat_hbm.at[cg].at[src_v.at[pl.ds(k * _G, _G)]],
                rows_v.at[b], gsems[b])

        def wait_gather(k, b):
            pltpu.make_async_copy(
                feat_hbm.at[cg].at[src_v.at[pl.ds(k * _G, _G)]],
                rows_v.at[b], gsems[b]).wait()

        def wait_scatter(b):
            pltpu.make_async_copy(sbuf_v.at[b], acc_sh.at[loc_v.at[b]],
                                  ssems[b]).wait()

        fire_gather(0, 0)
        fire_gather(1, 1)
        zeros16 = jnp.zeros((_L,), jnp.float32)
        pad16 = jnp.full((_L,), _N, jnp.int32)
        for b in range(2):
            for q in range(_G // _L):
                loc_v[b, pl.ds(q * _L, _L)] = pad16
            for e in range(_G):
                for j in range(_DG // _L):
                    sbuf_v[b, e, pl.ds(j * _L, _L)] = zeros16
            pltpu.async_copy(sbuf_v.at[b], acc_sh.at[loc_v.at[b]],
                             ssems[b], add=True)

        def process(k, b):
            wait_gather(k, b)
            wait_scatter(b)
            for q in range(_G // _L):
                sl = pl.ds(k * _G + q * _L, _L)
                loc_v[b, pl.ds(q * _L, _L)] = dst_v[sl]
                cs_v[pl.ds(q * _L, _L)] = w_v[sl]
            for e in range(_G):
                bc = plsc.load_gather(cs_v, [jnp.full((_L,), e, jnp.int32)])
                for j in range(_DG // _L):
                    sl = pl.ds(j * _L, _L)
                    sbuf_v[b, e, sl] = rows_v[b, e, sl] * bc
            kn = jnp.minimum(k + 2, _NCH - 1)
            fire_gather(kn, b)
            pltpu.async_copy(sbuf_v.at[b], acc_sh.at[loc_v.at[b]],
                             ssems[b], add=True)

        def iter_body(it, carry):
            process(it * 2, 0)
            process(it * 2 + 1, 1)
            return carry

        lax.fori_loop(0, _NCH // 2, iter_body, 0)

        for b in range(2):
            wait_gather(_NCH - 1, b)
            wait_scatter(b)

        plsc.subcore_barrier()
        pltpu.sync_copy(acc_sh.at[pl.ds(s * _RPT, _RPT)],
                        out_hbm.at[pl.ds(s * _RPT, _RPT), pl.ds(col, _DG)])
        plsc.subcore_barrier()


def _mm_body(relu, split_out, a_ref, x4_ref, hist_ref, wl_ref, wr_ref, o_ref):
    deg = jnp.sum(hist_ref[...], axis=1)
    dsc = 1.0 / jnp.square(jnp.maximum(deg, 1.0))
    a = a_ref[...] * dsc[:, None]
    ab = a.astype(jnp.bfloat16)
    x4 = x4_ref[...]
    x = jnp.concatenate([x4[g] for g in range(_CG)], axis=1)
    xb = x.astype(jnp.bfloat16)
    dn = (((1,), (1,)), ((), ()))
    acc = lax.dot_general(ab, wl_ref[...].astype(jnp.bfloat16), dn,
                          preferred_element_type=jnp.float32)
    acc = acc + lax.dot_general(xb, wr_ref[...].astype(jnp.bfloat16), dn,
                                preferred_element_type=jnp.float32)
    if relu:
        acc = jnp.maximum(acc, 0.0)
    if split_out:
        for g in range(_CG):
            o_ref[g] = acc[:, g * _DG:(g + 1) * _DG]
    else:
        o_ref[...] = acc


def _mm(a, x4, hists, wl, wr, relu, split_out):
    blk = 1000
    if split_out:
        out_shape = jax.ShapeDtypeStruct((_CG, _N, _DG), jnp.float32)
        out_spec = pl.BlockSpec((_CG, blk, _DG), lambda i: (0, i, 0))
    else:
        out_shape = jax.ShapeDtypeStruct((_N, _D), jnp.float32)
        out_spec = pl.BlockSpec((blk, _D), lambda i: (i, 0))
    return pl.pallas_call(
        functools.partial(_mm_body, relu, split_out),
        grid=(_N // blk,),
        in_specs=[
            pl.BlockSpec((blk, _D), lambda i: (i, 0)),
            pl.BlockSpec((_CG, blk, _DG), lambda i: (0, i, 0)),
            pl.BlockSpec((blk, _NW), lambda i: (i, 0)),
            pl.BlockSpec((_D, _D), lambda i: (0, 0)),
            pl.BlockSpec((_D, _D), lambda i: (0, 0)),
        ],
        out_specs=out_spec,
        out_shape=out_shape,
    )(a, x4, hists, wl, wr)


def kernel(x, edge_index, edge_weight, W1l, W1r, W2l, W2r):
    src = edge_index[0]
    dst = edge_index[1]
    pad = _EPAD - _E
    src_p = jnp.concatenate([src, jnp.zeros((pad,), jnp.int32)])
    dst_p = jnp.concatenate([dst, jnp.full((pad,), _N, jnp.int32)])
    w_p = jnp.concatenate([edge_weight, jnp.zeros((pad,), jnp.float32)])
    zeros = jnp.zeros((_RPT, _DG), jnp.float32)

    hists = _hist_kernel(dst_p)[:, :_N].T

    x4 = x.reshape(_N, _CG, _DG).transpose(1, 0, 2)
    agg1 = _agg_kernel(x4, src_p, dst_p, w_p, zeros)[:_N]
    h4 = _mm(agg1, x4, hists, W1l, W1r, True, True)

    agg2 = _agg_kernel(h4, src_p, dst_p, w_p, zeros)[:_N]
    out = _mm(agg2, h4, hists, W2l, W2r, False, False)
    return out


# speedup vs baseline: 1.0259x; 1.0259x over previous
import functools

import jax
import jax.numpy as jnp
from jax import lax
from jax.experimental import pallas as pl
from jax.experimental.pallas import tpu as pltpu
from jax.experimental.pallas import tpu_sc as plsc

_N = 10000
_E = 160000
_D = 256

_NC = 2
_NS = 16
_NW = _NC * _NS
_L = 16

_NPAD = 10240
_EPAD = 163840
_TE_A = _EPAD // _NW
_TE_B = _EPAD // _NS
_G = 64
_NB = 4
_NCH = _TE_B // _G
_CG = 4
_DG = _D // _CG
_RPT = _NPAD // _NS

_mesh = plsc.VectorSubcoreMesh(core_axis_name="c", subcore_axis_name="s")


@functools.partial(
    pl.kernel,
    out_type=jax.ShapeDtypeStruct((_NW, _NPAD), jnp.float32),
    mesh=_mesh,
    compiler_params=pltpu.CompilerParams(needs_layout_passes=False),
    scratch_types=[
        pltpu.VMEM((_NPAD,), jnp.float32),
        pltpu.VMEM((_TE_A,), jnp.int32),
    ],
)
def _hist_kernel(dst_hbm, out_hbm, hist_v, dst_v):
    c = lax.axis_index("c")
    s = lax.axis_index("s")
    wid = c * _NS + s

    zeros16 = jnp.zeros((_L,), jnp.float32)

    def zbody(i, carry):
        hist_v[pl.ds(i * _L, _L)] = zeros16
        return carry

    lax.fori_loop(0, _NPAD // _L, zbody, 0)

    pltpu.sync_copy(dst_hbm.at[pl.ds(wid * _TE_A, _TE_A)], dst_v)

    ones16 = jnp.ones((_L,), jnp.float32)

    def body(i, carry):
        idx = dst_v[pl.ds(i * _L, _L)]
        plsc.addupdate_scatter(hist_v, [idx], ones16)
        return carry

    lax.fori_loop(0, _TE_A // _L, body, 0)

    pltpu.sync_copy(hist_v, out_hbm.at[wid])


@functools.partial(
    pl.kernel,
    out_type=jax.ShapeDtypeStruct((_NPAD, _D), jnp.float32),
    mesh=_mesh,
    compiler_params=pltpu.CompilerParams(needs_layout_passes=False,
                                         use_tc_tiling_on_sc=False),
    scratch_types=[
        pltpu.VMEM((_TE_B,), jnp.int32),
        pltpu.VMEM((_TE_B,), jnp.int32),
        pltpu.VMEM((_TE_B,), jnp.float32),
        pltpu.VMEM((_NB, _G, _DG), jnp.float32),
        pltpu.VMEM((_NB, _G, _DG), jnp.float32),
        pltpu.VMEM((_G,), jnp.float32),
        pltpu.VMEM((_NB, _G), jnp.int32),
        pltpu.VMEM_SHARED((_NPAD, _DG), jnp.float32),
        [pltpu.SemaphoreType.DMA] * _NB,
        [pltpu.SemaphoreType.DMA] * _NB,
    ],
)
def _agg_kernel(feat_hbm, src_hbm, dst_hbm, w_hbm, zeros_hbm, out_hbm,
                src_v, dst_v, w_v, rows_v, sbuf_v, cs_v, loc_v, acc_sh,
                gsems, ssems):
    c = lax.axis_index("c")
    s = lax.axis_index("s")
    base_e = s * _TE_B

    pltpu.sync_copy(src_hbm.at[pl.ds(base_e, _TE_B)], src_v)
    pltpu.sync_copy(dst_hbm.at[pl.ds(base_e, _TE_B)], dst_v)
    pltpu.sync_copy(w_hbm.at[pl.ds(base_e, _TE_B)], w_v)

    for p in range(2):
        cg = c * 2 + p
        col = cg * _DG

        pltpu.sync_copy(zeros_hbm, acc_sh.at[pl.ds(s * _RPT, _RPT)])
        plsc.subcore_barrier()

        def fire_gather(k, b):
            pltpu.async_copy(
                feat_hbm.at[cg].at[src_v.at[pl.ds(k * _G, _G)]],
                rows_v.at[b], gsems[b])

        def wait_gather(k, b):
            pltpu.make_async_copy(
                feat_hbm.at[cg].at[src_v.at[pl.ds(k * _G, _G)]],
                rows_v.at[b], gsems[b]).wait()

        def wait_scatter(b):
            pltpu.make_async_copy(sbuf_v.at[b], acc_sh.at[loc_v.at[b]],
                                  ssems[b]).wait()

        for b in range(_NB):
            fire_gather(b, b)
        zeros16 = jnp.zeros((_L,), jnp.float32)
        pad16 = jnp.full((_L,), _N, jnp.int32)
        for b in range(_NB):
            for q in range(_G // _L):
                loc_v[b, pl.ds(q * _L, _L)] = pad16
            for e in range(_G):
                for j in range(_DG // _L):
                    sbuf_v[b, e, pl.ds(j * _L, _L)] = zeros16
            pltpu.async_copy(sbuf_v.at[b], acc_sh.at[loc_v.at[b]],
                             ssems[b], add=True)

        def process(k, b):
            wait_gather(k, b)
            wait_scatter(b)
            for q in range(_G // _L):
                sl = pl.ds(k * _G + q * _L, _L)
                loc_v[b, pl.ds(q * _L, _L)] = dst_v[sl]
                cs_v[pl.ds(q * _L, _L)] = w_v[sl]
            for e in range(_G):
                bc = plsc.load_gather(cs_v, [jnp.full((_L,), e, jnp.int32)])
                for j in range(_DG // _L):
                    sl = pl.ds(j * _L, _L)
                    sbuf_v[b, e, sl] = rows_v[b, e, sl] * bc
            kn = jnp.minimum(k + _NB, _NCH - 1)
            fire_gather(kn, b)
            pltpu.async_copy(sbuf_v.at[b], acc_sh.at[loc_v.at[b]],
                             ssems[b], add=True)

        def iter_body(it, carry):
            for b in range(_NB):
                process(it * _NB + b, b)
            return carry

        lax.fori_loop(0, _NCH // _NB, iter_body, 0)

        for b in range(_NB):
            wait_gather(_NCH - 1, b)
            wait_scatter(b)

        plsc.subcore_barrier()
        pltpu.sync_copy(acc_sh.at[pl.ds(s * _RPT, _RPT)],
                        out_hbm.at[pl.ds(s * _RPT, _RPT), pl.ds(col, _DG)])
        plsc.subcore_barrier()


def _mm_body(relu, split_out, a_ref, x4_ref, hist_ref, wl_ref, wr_ref, o_ref):
    deg = jnp.sum(hist_ref[...], axis=1)
    dsc = 1.0 / jnp.square(jnp.maximum(deg, 1.0))
    a = a_ref[...] * dsc[:, None]
    ab = a.astype(jnp.bfloat16)
    x4 = x4_ref[...]
    x = jnp.concatenate([x4[g] for g in range(_CG)], axis=1)
    xb = x.astype(jnp.bfloat16)
    dn = (((1,), (1,)), ((), ()))
    acc = lax.dot_general(ab, wl_ref[...].astype(jnp.bfloat16), dn,
                          preferred_element_type=jnp.float32)
    acc = acc + lax.dot_general(xb, wr_ref[...].astype(jnp.bfloat16), dn,
                                preferred_element_type=jnp.float32)
    if relu:
        acc = jnp.maximum(acc, 0.0)
    if split_out:
        for g in range(_CG):
            o_ref[g] = acc[:, g * _DG:(g + 1) * _DG]
    else:
        o_ref[...] = acc


def _mm(a, x4, hists, wl, wr, relu, split_out):
    blk = 1000
    if split_out:
        out_shape = jax.ShapeDtypeStruct((_CG, _N, _DG), jnp.float32)
        out_spec = pl.BlockSpec((_CG, blk, _DG), lambda i: (0, i, 0))
    else:
        out_shape = jax.ShapeDtypeStruct((_N, _D), jnp.float32)
        out_spec = pl.BlockSpec((blk, _D), lambda i: (i, 0))
    return pl.pallas_call(
        functools.partial(_mm_body, relu, split_out),
        grid=(_N // blk,),
        in_specs=[
            pl.BlockSpec((blk, _D), lambda i: (i, 0)),
            pl.BlockSpec((_CG, blk, _DG), lambda i: (0, i, 0)),
            pl.BlockSpec((blk, _NW), lambda i: (i, 0)),
            pl.BlockSpec((_D, _D), lambda i: (0, 0)),
            pl.BlockSpec((_D, _D), lambda i: (0, 0)),
        ],
        out_specs=out_spec,
        out_shape=out_shape,
    )(a, x4, hists, wl, wr)


def kernel(x, edge_index, edge_weight, W1l, W1r, W2l, W2r):
    src = edge_index[0]
    dst = edge_index[1]
    pad = _EPAD - _E
    src_p = jnp.concatenate([src, jnp.zeros((pad,), jnp.int32)])
    dst_p = jnp.concatenate([dst, jnp.full((pad,), _N, jnp.int32)])
    w_p = jnp.concatenate([edge_weight, jnp.zeros((pad,), jnp.float32)])
    zeros = jnp.zeros((_RPT, _DG), jnp.float32)

    hists = _hist_kernel(dst_p)[:, :_N].T

    x4 = x.reshape(_N, _CG, _DG).transpose(1, 0, 2)
    agg1 = _agg_kernel(x4, src_p, dst_p, w_p, zeros)[:_N]
    h4 = _mm(agg1, x4, hists, W1l, W1r, True, True)

    agg2 = _agg_kernel(h4, src_p, dst_p, w_p, zeros)[:_N]
    out = _mm(agg2, h4, hists, W2l, W2r, False, False)
    return out


# speedup vs baseline: 1.1369x; 1.1082x over previous
import functools

import jax
import jax.numpy as jnp
from jax import lax
from jax.experimental import pallas as pl
from jax.experimental.pallas import tpu as pltpu
from jax.experimental.pallas import tpu_sc as plsc

_N = 10000
_E = 160000
_D = 256

_NC = 2
_NS = 16
_NW = _NC * _NS
_L = 16

_NPAD = 10240
_EPAD = 163840
_TE_A = _EPAD // _NW
_TE_B = _EPAD // _NS
_G = 64
_NB = 2
_NCH = _TE_B // _G
_CG = 4
_DG = _D // _CG
_RPT = _NPAD // _NS

_mesh = plsc.VectorSubcoreMesh(core_axis_name="c", subcore_axis_name="s")


@functools.partial(
    pl.kernel,
    out_type=jax.ShapeDtypeStruct((_NW, _NPAD), jnp.float32),
    mesh=_mesh,
    compiler_params=pltpu.CompilerParams(needs_layout_passes=False),
    scratch_types=[
        pltpu.VMEM((_NPAD,), jnp.float32),
        pltpu.VMEM((_TE_A,), jnp.int32),
    ],
)
def _hist_kernel(dst_hbm, out_hbm, hist_v, dst_v):
    c = lax.axis_index("c")
    s = lax.axis_index("s")
    wid = c * _NS + s

    zeros16 = jnp.zeros((_L,), jnp.float32)

    def zbody(i, carry):
        hist_v[pl.ds(i * _L, _L)] = zeros16
        return carry

    lax.fori_loop(0, _NPAD // _L, zbody, 0)

    pltpu.sync_copy(dst_hbm.at[pl.ds(wid * _TE_A, _TE_A)], dst_v)

    ones16 = jnp.ones((_L,), jnp.float32)

    def body(i, carry):
        idx = dst_v[pl.ds(i * _L, _L)]
        plsc.addupdate_scatter(hist_v, [idx], ones16)
        return carry

    lax.fori_loop(0, _TE_A // _L, body, 0)

    pltpu.sync_copy(hist_v, out_hbm.at[wid])


@functools.partial(
    pl.kernel,
    out_type=jax.ShapeDtypeStruct((_NPAD, _D), jnp.float32),
    mesh=_mesh,
    compiler_params=pltpu.CompilerParams(needs_layout_passes=False,
                                         use_tc_tiling_on_sc=False),
    scratch_types=[
        pltpu.VMEM((_TE_B,), jnp.int32),
        pltpu.VMEM((_TE_B,), jnp.int32),
        pltpu.VMEM((_TE_B,), jnp.float32),
        pltpu.VMEM((_NB, _G, _DG), jnp.float32),
        pltpu.VMEM((_NB, _G, _DG), jnp.float32),
        pltpu.VMEM((_G,), jnp.float32),
        pltpu.VMEM((_NB, _G), jnp.int32),
        pltpu.VMEM_SHARED((_NPAD, _DG), jnp.float32),
        [pltpu.SemaphoreType.DMA] * _NB,
        [pltpu.SemaphoreType.DMA] * _NB,
    ],
)
def _agg_kernel(feat_hbm, src_hbm, dst_hbm, w_hbm, zeros_hbm, out_hbm,
                src_v, dst_v, w_v, rows_v, sbuf_v, cs_v, loc_v, acc_sh,
                gsems, ssems):
    c = lax.axis_index("c")
    s = lax.axis_index("s")
    base_e = s * _TE_B

    pltpu.sync_copy(src_hbm.at[pl.ds(base_e, _TE_B)], src_v)
    pltpu.sync_copy(dst_hbm.at[pl.ds(base_e, _TE_B)], dst_v)
    pltpu.sync_copy(w_hbm.at[pl.ds(base_e, _TE_B)], w_v)

    for p in range(2):
        cg = c * 2 + p
        col = cg * _DG

        pltpu.sync_copy(zeros_hbm, acc_sh.at[pl.ds(s * _RPT, _RPT)])
        plsc.subcore_barrier()

        def fire_gather(k, b):
            pltpu.async_copy(
                feat_hbm.at[cg].at[src_v.at[pl.ds(k * _G, _G)]],
                rows_v.at[b], gsems[b])

        def wait_gather(k, b):
            pltpu.make_async_copy(
                feat_hbm.at[cg].at[src_v.at[pl.ds(k * _G, _G)]],
                rows_v.at[b], gsems[b]).wait()

        def wait_scatter(b):
            pltpu.make_async_copy(sbuf_v.at[b], acc_sh.at[loc_v.at[b]],
                                  ssems[b]).wait()

        for b in range(_NB):
            fire_gather(b, b)
        zeros16 = jnp.zeros((_L,), jnp.float32)
        pad16 = jnp.full((_L,), _N, jnp.int32)
        for b in range(_NB):
            for q in range(_G // _L):
                loc_v[b, pl.ds(q * _L, _L)] = pad16
            for e in range(_G):
                for j in range(_DG // _L):
                    sbuf_v[b, e, pl.ds(j * _L, _L)] = zeros16
            pltpu.async_copy(sbuf_v.at[b], acc_sh.at[loc_v.at[b]],
                             ssems[b], add=True)

        def process(k, b):
            wait_gather(k, b)
            wait_scatter(b)
            for q in range(_G // _L):
                sl = pl.ds(k * _G + q * _L, _L)
                loc_v[b, pl.ds(q * _L, _L)] = dst_v[sl]
                cs_v[pl.ds(q * _L, _L)] = w_v[sl]
            for e in range(_G):
                bc = plsc.load_gather(cs_v, [jnp.full((_L,), e, jnp.int32)])
                for j in range(_DG // _L):
                    sl = pl.ds(j * _L, _L)
                    sbuf_v[b, e, sl] = rows_v[b, e, sl] * bc
            kn = jnp.minimum(k + _NB, _NCH - 1)
            fire_gather(kn, b)
            pltpu.async_copy(sbuf_v.at[b], acc_sh.at[loc_v.at[b]],
                             ssems[b], add=True)

        def iter_body(it, carry):
            for b in range(_NB):
                process(it * _NB + b, b)
            return carry

        lax.fori_loop(0, _NCH // _NB, iter_body, 0)

        for b in range(_NB):
            wait_gather(_NCH - 1, b)
            wait_scatter(b)

        plsc.subcore_barrier()
        pltpu.sync_copy(acc_sh.at[pl.ds(s * _RPT, _RPT)],
                        out_hbm.at[pl.ds(s * _RPT, _RPT), pl.ds(col, _DG)])
        plsc.subcore_barrier()


def _mm_body(relu, split_out, a_ref, x4_ref, hist_ref, wl_ref, wr_ref, o_ref):
    deg = jnp.sum(hist_ref[...], axis=1)
    dsc = 1.0 / jnp.square(jnp.maximum(deg, 1.0))
    a = a_ref[...] * dsc[:, None]
    ab = a.astype(jnp.bfloat16)
    x4 = x4_ref[...]
    x = jnp.concatenate([x4[g] for g in range(_CG)], axis=1)
    xb = x.astype(jnp.bfloat16)
    dn = (((1,), (1,)), ((), ()))
    acc = lax.dot_general(ab, wl_ref[...].astype(jnp.bfloat16), dn,
                          preferred_element_type=jnp.float32)
    acc = acc + lax.dot_general(xb, wr_ref[...].astype(jnp.bfloat16), dn,
                                preferred_element_type=jnp.float32)
    if relu:
        acc = jnp.maximum(acc, 0.0)
    if split_out:
        for g in range(_CG):
            o_ref[g] = acc[:, g * _DG:(g + 1) * _DG]
    else:
        o_ref[...] = acc


def _mm(a, x4, hists, wl, wr, relu, split_out):
    blk = 1000
    if split_out:
        out_shape = jax.ShapeDtypeStruct((_CG, _N, _DG), jnp.float32)
        out_spec = pl.BlockSpec((_CG, blk, _DG), lambda i: (0, i, 0))
    else:
        out_shape = jax.ShapeDtypeStruct((_N, _D), jnp.float32)
        out_spec = pl.BlockSpec((blk, _D), lambda i: (i, 0))
    return pl.pallas_call(
        functools.partial(_mm_body, relu, split_out),
        grid=(_N // blk,),
        in_specs=[
            pl.BlockSpec((blk, _D), lambda i: (i, 0)),
            pl.BlockSpec((_CG, blk, _DG), lambda i: (0, i, 0)),
            pl.BlockSpec((blk, _NW), lambda i: (i, 0)),
            pl.BlockSpec((_D, _D), lambda i: (0, 0)),
            pl.BlockSpec((_D, _D), lambda i: (0, 0)),
        ],
        out_specs=out_spec,
        out_shape=out_shape,
    )(a, x4, hists, wl, wr)


def kernel(x, edge_index, edge_weight, W1l, W1r, W2l, W2r):
    src = edge_index[0]
    dst = edge_index[1]
    pad = _EPAD - _E
    src_p = jnp.concatenate([src, jnp.zeros((pad,), jnp.int32)])
    dst_p = jnp.concatenate([dst, jnp.full((pad,), _N, jnp.int32)])
    w_p = jnp.concatenate([edge_weight, jnp.zeros((pad,), jnp.float32)])
    zeros = jnp.zeros((_RPT, _DG), jnp.float32)

    hists = _hist_kernel(dst_p)[:, :_N].T

    x4 = x.reshape(_N, _CG, _DG).transpose(1, 0, 2)
    agg1 = _agg_kernel(x4, src_p, dst_p, w_p, zeros)[:_N]
    h4 = _mm(agg1, x4, hists, W1l, W1r, True, True)

    agg2 = _agg_kernel(h4, src_p, dst_p, w_p, zeros)[:_N]
    out = _mm(agg2, h4, hists, W2l, W2r, False, False)
    return out


# speedup vs baseline: 1.3940x; 1.2261x over previous
import functools

import jax
import jax.numpy as jnp
from jax import lax
from jax.experimental import pallas as pl
from jax.experimental.pallas import tpu as pltpu
from jax.experimental.pallas import tpu_sc as plsc

_N = 10000
_E = 160000
_D = 256

_NC = 2
_NS = 16
_NW = _NC * _NS
_L = 16

_NPAD = 10240
_EPAD = 161792
_TE_A = _EPAD // _NW
_TE_B = _EPAD // _NS
_G = 64
_NB = 2
_NCH = _TE_B // _G
_CG = 4
_DG = _D // _CG
_RPT = _NPAD // _NS

_mesh = plsc.VectorSubcoreMesh(core_axis_name="c", subcore_axis_name="s")


@functools.partial(
    pl.kernel,
    out_type=jax.ShapeDtypeStruct((_NW, _NPAD), jnp.float32),
    mesh=_mesh,
    compiler_params=pltpu.CompilerParams(needs_layout_passes=False),
    scratch_types=[
        pltpu.VMEM((_NPAD,), jnp.float32),
        pltpu.VMEM((_TE_A,), jnp.int32),
    ],
)
def _hist_kernel(dst_hbm, out_hbm, hist_v, dst_v):
    c = lax.axis_index("c")
    s = lax.axis_index("s")
    wid = c * _NS + s

    zeros16 = jnp.zeros((_L,), jnp.float32)

    def zbody(i, carry):
        hist_v[pl.ds(i * _L, _L)] = zeros16
        return carry

    lax.fori_loop(0, _NPAD // _L, zbody, 0)

    pltpu.sync_copy(dst_hbm.at[pl.ds(wid * _TE_A, _TE_A)], dst_v)

    ones16 = jnp.ones((_L,), jnp.float32)

    def body(i, carry):
        idx = dst_v[pl.ds(i * _L, _L)]
        plsc.addupdate_scatter(hist_v, [idx], ones16)
        return carry

    lax.fori_loop(0, _TE_A // _L, body, 0)

    pltpu.sync_copy(hist_v, out_hbm.at[wid])


@functools.partial(
    pl.kernel,
    out_type=jax.ShapeDtypeStruct((_NPAD, _D), jnp.float32),
    mesh=_mesh,
    compiler_params=pltpu.CompilerParams(needs_layout_passes=False,
                                         use_tc_tiling_on_sc=False),
    scratch_types=[
        pltpu.VMEM((_TE_B,), jnp.int32),
        pltpu.VMEM((_TE_B,), jnp.int32),
        pltpu.VMEM((_TE_B,), jnp.float32),
        pltpu.VMEM((_NB, _G, _DG), jnp.float32),
        pltpu.VMEM((_NB, _G, _DG), jnp.float32),
        pltpu.VMEM((_G,), jnp.float32),
        pltpu.VMEM((_NB, _G), jnp.int32),
        pltpu.VMEM_SHARED((_NPAD, _DG), jnp.float32),
        pltpu.SemaphoreType.DMA,
        pltpu.SemaphoreType.DMA,
        pltpu.SemaphoreType.DMA,
        pltpu.SemaphoreType.DMA,
    ],
)
def _agg_kernel(feat_hbm, src_hbm, dst_hbm, w_hbm, zeros_hbm, out_hbm,
                src_v, dst_v, w_v, rows_v, sbuf_v, cs_v, loc_v, acc_sh,
                gsem0, gsem1, ssem0, ssem1):
    c = lax.axis_index("c")
    s = lax.axis_index("s")
    base_e = s * _TE_B
    gsems = (gsem0, gsem1)
    ssems = (ssem0, ssem1)

    pltpu.sync_copy(src_hbm.at[pl.ds(base_e, _TE_B)], src_v)
    pltpu.sync_copy(dst_hbm.at[pl.ds(base_e, _TE_B)], dst_v)
    pltpu.sync_copy(w_hbm.at[pl.ds(base_e, _TE_B)], w_v)

    for p in range(2):
        cg = c * 2 + p
        col = cg * _DG

        pltpu.sync_copy(zeros_hbm, acc_sh.at[pl.ds(s * _RPT, _RPT)])
        plsc.subcore_barrier()

        def fire_gather(k, b):
            pltpu.async_copy(
                feat_hbm.at[cg].at[src_v.at[pl.ds(k * _G, _G)]],
                rows_v.at[b], gsems[b])

        def wait_gather(k, b):
            pltpu.make_async_copy(
                feat_hbm.at[cg].at[src_v.at[pl.ds(k * _G, _G)]],
                rows_v.at[b], gsems[b]).wait()

        def wait_scatter(b):
            pltpu.make_async_copy(sbuf_v.at[b], acc_sh.at[loc_v.at[b]],
                                  ssems[b]).wait()

        for b in range(_NB):
            fire_gather(b, b)
        zeros16 = jnp.zeros((_L,), jnp.float32)
        pad16 = jnp.full((_L,), _N, jnp.int32)
        for b in range(_NB):
            for q in range(_G // _L):
                loc_v[b, pl.ds(q * _L, _L)] = pad16
            for e in range(_G):
                for j in range(_DG // _L):
                    sbuf_v[b, e, pl.ds(j * _L, _L)] = zeros16
            pltpu.async_copy(sbuf_v.at[b], acc_sh.at[loc_v.at[b]],
                             ssems[b], add=True)

        def process(k, b):
            wait_gather(k, b)
            wait_scatter(b)
            for q in range(_G // _L):
                sl = pl.ds(k * _G + q * _L, _L)
                loc_v[b, pl.ds(q * _L, _L)] = dst_v[sl]
                cs_v[pl.ds(q * _L, _L)] = w_v[sl]
            for e in range(_G):
                bc = plsc.load_gather(cs_v, [jnp.full((_L,), e, jnp.int32)])
                for j in range(_DG // _L):
                    sl = pl.ds(j * _L, _L)
                    sbuf_v[b, e, sl] = rows_v[b, e, sl] * bc
            kn = jnp.minimum(k + _NB, _NCH - 1)
            fire_gather(kn, b)
            pltpu.async_copy(sbuf_v.at[b], acc_sh.at[loc_v.at[b]],
                             ssems[b], add=True)

        def iter_body(it, carry):
            for b in range(_NB):
                process(it * _NB + b, b)
            return carry

        lax.fori_loop(0, _NCH // _NB, iter_body, 0)

        for b in range(_NB):
            wait_gather(_NCH - 1, b)
            wait_scatter(b)

        plsc.subcore_barrier()
        pltpu.sync_copy(acc_sh.at[pl.ds(s * _RPT, _RPT)],
                        out_hbm.at[pl.ds(s * _RPT, _RPT), pl.ds(col, _DG)])
        plsc.subcore_barrier()


def _mm_body(relu, split_out, a_ref, x4_ref, hist_ref, wl_ref, wr_ref, o_ref):
    deg = jnp.sum(hist_ref[...], axis=1)
    dsc = 1.0 / jnp.square(jnp.maximum(deg, 1.0))
    a = a_ref[...] * dsc[:, None]
    ab = a.astype(jnp.bfloat16)
    x4 = x4_ref[...]
    x = jnp.concatenate([x4[g] for g in range(_CG)], axis=1)
    xb = x.astype(jnp.bfloat16)
    dn = (((1,), (1,)), ((), ()))
    acc = lax.dot_general(ab, wl_ref[...].astype(jnp.bfloat16), dn,
                          preferred_element_type=jnp.float32)
    acc = acc + lax.dot_general(xb, wr_ref[...].astype(jnp.bfloat16), dn,
                                preferred_element_type=jnp.float32)
    if relu:
        acc = jnp.maximum(acc, 0.0)
    if split_out:
        for g in range(_CG):
            o_ref[g] = acc[:, g * _DG:(g + 1) * _DG]
    else:
        o_ref[...] = acc


def _mm(a, x4, hists, wl, wr, relu, split_out):
    blk = 1000
    if split_out:
        out_shape = jax.ShapeDtypeStruct((_CG, _N, _DG), jnp.float32)
        out_spec = pl.BlockSpec((_CG, blk, _DG), lambda i: (0, i, 0))
    else:
        out_shape = jax.ShapeDtypeStruct((_N, _D), jnp.float32)
        out_spec = pl.BlockSpec((blk, _D), lambda i: (i, 0))
    return pl.pallas_call(
        functools.partial(_mm_body, relu, split_out),
        grid=(_N // blk,),
        in_specs=[
            pl.BlockSpec((blk, _D), lambda i: (i, 0)),
            pl.BlockSpec((_CG, blk, _DG), lambda i: (0, i, 0)),
            pl.BlockSpec((blk, _NW), lambda i: (i, 0)),
            pl.BlockSpec((_D, _D), lambda i: (0, 0)),
            pl.BlockSpec((_D, _D), lambda i: (0, 0)),
        ],
        out_specs=out_spec,
        out_shape=out_shape,
    )(a, x4, hists, wl, wr)


def kernel(x, edge_index, edge_weight, W1l, W1r, W2l, W2r):
    src = edge_index[0]
    dst = edge_index[1]
    pad = _EPAD - _E
    src_p = jnp.concatenate([src, jnp.zeros((pad,), jnp.int32)])
    dst_p = jnp.concatenate([dst, jnp.full((pad,), _N, jnp.int32)])
    w_p = jnp.concatenate([edge_weight, jnp.zeros((pad,), jnp.float32)])
    zeros = jnp.zeros((_RPT, _DG), jnp.float32)

    hists = _hist_kernel(dst_p)[:, :_N].T

    x4 = x.reshape(_N, _CG, _DG).transpose(1, 0, 2)
    agg1 = _agg_kernel(x4, src_p, dst_p, w_p, zeros)[:_N]
    h4 = _mm(agg1, x4, hists, W1l, W1r, True, True)

    agg2 = _agg_kernel(h4, src_p, dst_p, w_p, zeros)[:_N]
    out = _mm(agg2, h4, hists, W2l, W2r, False, False)
    return out


# speedup vs baseline: 1.5823x; 1.1351x over previous
import functools

import jax
import jax.numpy as jnp
from jax import lax
from jax.experimental import pallas as pl
from jax.experimental.pallas import tpu as pltpu
from jax.experimental.pallas import tpu_sc as plsc

_N = 10000
_E = 160000
_D = 256

_NC = 2
_NS = 16
_NW = _NC * _NS
_L = 16

_NPAD = 10240
_EPAD = 161792
_TE_A = _EPAD // _NW
_TE_B = _EPAD // _NS
_G = 64
_NB = 2
_NCH = _TE_B // _G
_CG = 4
_DG = _D // _CG
_RPT = _NPAD // _NS
_PERM = [g * 64 + t * 32 + h * 16 + i
         for g in range(_CG) for t in range(2)
         for i in range(_L) for h in range(2)]

_mesh = plsc.VectorSubcoreMesh(core_axis_name="c", subcore_axis_name="s")


@functools.partial(
    pl.kernel,
    out_type=(jax.ShapeDtypeStruct((_NW, _NPAD), jnp.float32),
              jax.ShapeDtypeStruct((_NPAD, _D), jnp.float32)),
    mesh=_mesh,
    compiler_params=pltpu.CompilerParams(needs_layout_passes=False,
                                         use_tc_tiling_on_sc=False),
    scratch_types=[
        pltpu.VMEM((_TE_B,), jnp.int32),
        pltpu.VMEM((_TE_B,), jnp.int32),
        pltpu.VMEM((_TE_B,), jnp.float32),
        pltpu.VMEM((_NB, _G, _DG), jnp.bfloat16),
        pltpu.VMEM((_NB, _G, _DG), jnp.float32),
        pltpu.VMEM((_G,), jnp.float32),
        pltpu.VMEM((_NB, _G), jnp.int32),
        pltpu.VMEM((_NPAD,), jnp.float32),
        pltpu.VMEM_SHARED((_NPAD, _DG), jnp.float32),
        pltpu.SemaphoreType.DMA,
        pltpu.SemaphoreType.DMA,
        pltpu.SemaphoreType.DMA,
        pltpu.SemaphoreType.DMA,
    ],
)
def _agg_kernel(feat_hbm, src_hbm, dst_hbm, w_hbm, zeros_hbm,
                hist_hbm, out_hbm,
                src_v, dst_v, w_v, rows_v, sbuf_v, cs_v, loc_v, hist_v,
                acc_sh, gsem0, gsem1, ssem0, ssem1):
    c = lax.axis_index("c")
    s = lax.axis_index("s")
    base_e = s * _TE_B
    gsems = (gsem0, gsem1)
    ssems = (ssem0, ssem1)

    pltpu.sync_copy(src_hbm.at[pl.ds(base_e, _TE_B)], src_v)
    pltpu.sync_copy(dst_hbm.at[pl.ds(base_e, _TE_B)], dst_v)
    pltpu.sync_copy(w_hbm.at[pl.ds(base_e, _TE_B)], w_v)

    zs16 = jnp.zeros((_L,), jnp.float32)

    def hzbody(i, carry):
        hist_v[pl.ds(i * _L, _L)] = zs16
        return carry

    lax.fori_loop(0, _NPAD // _L, hzbody, 0)
    ones16 = jnp.ones((_L,), jnp.float32)
    hbase = c * _TE_A

    def hbody(i, carry):
        idx = dst_v[pl.ds(hbase + i * _L, _L)]
        plsc.addupdate_scatter(hist_v, [idx], ones16)
        return carry

    lax.fori_loop(0, _TE_A // _L, hbody, 0)
    pltpu.sync_copy(hist_v, hist_hbm.at[c * _NS + s])

    for p in range(2):
        cg = c * 2 + p
        col = cg * _DG

        pltpu.sync_copy(zeros_hbm, acc_sh.at[pl.ds(s * _RPT, _RPT)])
        plsc.subcore_barrier()

        def fire_gather(k, b):
            pltpu.async_copy(
                feat_hbm.at[cg].at[src_v.at[pl.ds(k * _G, _G)]],
                rows_v.at[b], gsems[b])

        def wait_gather(k, b):
            pltpu.make_async_copy(
                feat_hbm.at[cg].at[src_v.at[pl.ds(k * _G, _G)]],
                rows_v.at[b], gsems[b]).wait()

        def wait_scatter(b):
            pltpu.make_async_copy(sbuf_v.at[b], acc_sh.at[loc_v.at[b]],
                                  ssems[b]).wait()

        for b in range(_NB):
            fire_gather(b, b)
        zeros16 = jnp.zeros((_L,), jnp.float32)
        pad16 = jnp.full((_L,), _N, jnp.int32)
        for b in range(_NB):
            for q in range(_G // _L):
                loc_v[b, pl.ds(q * _L, _L)] = pad16
            for e in range(_G):
                for j in range(_DG // _L):
                    sbuf_v[b, e, pl.ds(j * _L, _L)] = zeros16
            pltpu.async_copy(sbuf_v.at[b], acc_sh.at[loc_v.at[b]],
                             ssems[b], add=True)

        def process(k, b):
            wait_gather(k, b)
            wait_scatter(b)
            for q in range(_G // _L):
                sl = pl.ds(k * _G + q * _L, _L)
                loc_v[b, pl.ds(q * _L, _L)] = dst_v[sl]
                cs_v[pl.ds(q * _L, _L)] = w_v[sl]
            for e in range(_G):
                bc = plsc.load_gather(cs_v, [jnp.full((_L,), e, jnp.int32)])
                for j in range(_DG // (2 * _L)):
                    ab = rows_v[b, e, pl.ds(j * 2 * _L, 2 * _L)]
                    va, vb = plsc.unpack(ab, format=plsc.PackFormat.INTERLEAVED)
                    sbuf_v[b, e, pl.ds(j * 2 * _L, _L)] = va * bc
                    sbuf_v[b, e, pl.ds(j * 2 * _L + _L, _L)] = vb * bc
            kn = jnp.minimum(k + _NB, _NCH - 1)
            fire_gather(kn, b)
            pltpu.async_copy(sbuf_v.at[b], acc_sh.at[loc_v.at[b]],
                             ssems[b], add=True)

        def iter_body(it, carry):
            for b in range(_NB):
                process(it * _NB + b, b)
            return carry

        lax.fori_loop(0, _NCH // _NB, iter_body, 0)

        for b in range(_NB):
            wait_gather(_NCH - 1, b)
            wait_scatter(b)

        plsc.subcore_barrier()
        pltpu.sync_copy(acc_sh.at[pl.ds(s * _RPT, _RPT)],
                        out_hbm.at[pl.ds(s * _RPT, _RPT), pl.ds(col, _DG)])
        plsc.subcore_barrier()


def _mmx_body(x4_ref, wr_ref, o_ref):
    x4 = x4_ref[...]
    xb = jnp.concatenate([x4[g] for g in range(_CG)], axis=1)
    dn = (((1,), (1,)), ((), ()))
    o_ref[...] = lax.dot_general(xb, wr_ref[...].astype(jnp.bfloat16), dn,
                                 preferred_element_type=jnp.float32)


def _mmx(x4, wr):
    blk = 1000
    return pl.pallas_call(
        _mmx_body,
        grid=(_N // blk,),
        in_specs=[
            pl.BlockSpec((_CG, blk, _DG), lambda i: (0, i, 0)),
            pl.BlockSpec((_D, _D), lambda i: (0, 0)),
        ],
        out_specs=pl.BlockSpec((blk, _D), lambda i: (i, 0)),
        out_shape=jax.ShapeDtypeStruct((_N, _D), jnp.float32),
    )(x4, wr)


def _mma_body(relu, split_out, a_ref, xr_ref, hist_ref, wl_ref, o_ref):
    deg = jnp.sum(hist_ref[...], axis=1)
    dsc = 1.0 / jnp.square(jnp.maximum(deg, 1.0))
    a = a_ref[...] * dsc[:, None]
    ab = a.astype(jnp.bfloat16)
    dn = (((1,), (1,)), ((), ()))
    acc = lax.dot_general(ab, wl_ref[...].astype(jnp.bfloat16), dn,
                          preferred_element_type=jnp.float32)
    acc = acc + xr_ref[...]
    if relu:
        acc = jnp.maximum(acc, 0.0)
    if split_out:
        for g in range(_CG):
            o_ref[g] = acc[:, g * _DG:(g + 1) * _DG].astype(jnp.bfloat16)
    else:
        o_ref[...] = acc


def _mma(a, xr, hists, wl, relu, split_out):
    blk = 1000
    if split_out:
        out_shape = jax.ShapeDtypeStruct((_CG, _N, _DG), jnp.bfloat16)
        out_spec = pl.BlockSpec((_CG, blk, _DG), lambda i: (0, i, 0))
    else:
        out_shape = jax.ShapeDtypeStruct((_N, _D), jnp.float32)
        out_spec = pl.BlockSpec((blk, _D), lambda i: (i, 0))
    return pl.pallas_call(
        functools.partial(_mma_body, relu, split_out),
        grid=(_N // blk,),
        in_specs=[
            pl.BlockSpec((blk, _D), lambda i: (i, 0)),
            pl.BlockSpec((blk, _D), lambda i: (i, 0)),
            pl.BlockSpec((blk, _NW), lambda i: (i, 0)),
            pl.BlockSpec((_D, _D), lambda i: (0, 0)),
        ],
        out_specs=out_spec,
        out_shape=out_shape,
    )(a, xr, hists, wl)


def kernel(x, edge_index, edge_weight, W1l, W1r, W2l, W2r):
    src = edge_index[0]
    dst = edge_index[1]
    pad = _EPAD - _E
    src_p = jnp.concatenate([src, jnp.zeros((pad,), jnp.int32)])
    dst_p = jnp.concatenate([dst, jnp.full((pad,), _N, jnp.int32)])
    w_p = jnp.concatenate([edge_weight, jnp.zeros((pad,), jnp.float32)])
    zeros = jnp.zeros((_RPT, _DG), jnp.float32)

    perm = jnp.asarray(_PERM)
    x4 = x.reshape(_N, _CG, _DG).transpose(1, 0, 2)
    x4 = x4.reshape(_CG, _N, 2, 2, _L).transpose(0, 1, 2, 4, 3)
    x4 = x4.reshape(_CG, _N, _DG).astype(jnp.bfloat16)
    W1l_p = W1l[perm]
    W1r_p = W1r[perm][:, perm]
    W2l_p = W2l
    W2r_p = W2r[:, perm]

    xr1 = _mmx(x4, W1r_p)
    hists_p, agg1 = _agg_kernel(x4, src_p, dst_p, w_p, zeros)
    hists = hists_p[:, :_N].T
    h4 = _mma(agg1[:_N], xr1, hists, W1l_p, True, True)

    xr2 = _mmx(h4, W2r_p)
    _, agg2 = _agg_kernel(h4, src_p, dst_p, w_p, zeros)
    out = _mma(agg2[:_N], xr2, hists, W2l_p, False, False)
    return out
